# baseline (speedup 1.0000x reference)
import functools

import jax
import jax.numpy as jnp
from jax import lax
from jax.experimental import pallas as pl
from jax.experimental.pallas import tpu as pltpu
from jax.experimental.pallas import tpu_sc as plsc

_VOCAB = 1000000
_D = 64
_B = 4096
_S = 200

_NC, _NS = 2, 16
_NW = _NC * _NS
_BW = _B // _NW
_TB = _B // 128
_TD = _D // 8
_OUT_R = _B * _S * _D // 128


def _pos_encoding():
    pos = jnp.arange(_S, dtype=jnp.float32)[:, None]
    i = jnp.arange(_D)[None, :]
    angle_rates = 1.0 / jnp.power(10000.0, (2.0 * (i // 2)) / jnp.float32(_D))
    angle_rads = pos * angle_rates
    return jnp.where(i % 2 == 0, jnp.sin(angle_rads), jnp.cos(angle_rads)).astype(
        jnp.float32
    )


@functools.partial(
    pl.kernel,
    out_type=jax.ShapeDtypeStruct((_OUT_R, 128), jnp.float32),
    mesh=plsc.VectorSubcoreMesh(core_axis_name="c", subcore_axis_name="s"),
    scratch_types=[
        pltpu.VMEM((_BW, _S), jnp.int32),
        pltpu.VMEM((_S, _BW), jnp.int32),
        pltpu.VMEM((_S, _D), jnp.float32),
        pltpu.VMEM((_BW, _D), jnp.float32),
        pltpu.VMEM((_BW, _D), jnp.float32),
        pltpu.VMEM((_D, _BW), jnp.float32),
        pltpu.VMEM((_D, _BW), jnp.float32),
        pltpu.SemaphoreType.DMA,
        pltpu.SemaphoreType.DMA,
        pltpu.SemaphoreType.DMA,
        pltpu.SemaphoreType.DMA,
    ],
    compiler_params=pltpu.CompilerParams(
        use_tc_tiling_on_sc=False, needs_layout_passes=False
    ),
)
def _embed(
    x_hbm, table_hbm, pe_hbm, out_hbm,
    idx_v, idxt_v, pe_v, bg0, bg1, bo0, bo1, sg0, sg1, so0, so1,
):
    w = lax.axis_index("s") * _NC + lax.axis_index("c")
    pltpu.sync_copy(x_hbm.at[pl.ds(w * _BW, _BW)], idx_v)
    pltpu.sync_copy(pe_hbm, pe_v)

    lane = lax.iota(jnp.int32, 16)
    biota = [lane + 16 * j for j in range(_BW // 16)]

    def tr_idx(s, carry):
        col = jnp.broadcast_to(s, (16,)).astype(jnp.int32)
        for j in range(_BW // 16):
            idxt_v[s, pl.ds(16 * j, 16)] = plsc.load_gather(idx_v, [biota[j], col])
        return carry

    lax.fori_loop(0, _S, tr_idx, 0)

    def start_gather(s, bg, sg):
        pltpu.async_copy(table_hbm.at[idxt_v.at[s]], bg, sg)

    def wait_gather(bg, sg):
        pltpu.make_async_copy(table_hbm.at[pl.ds(0, _BW)], bg, sg).wait()

    def start_out(s, bo, so):
        for td in range(_TD):
            pltpu.async_copy(
                bo.at[pl.ds(td * 8, 8)],
                out_hbm.at[pl.ds(((s * _TD + td) * _TB + w) * 8, 8)],
                so,
            )

    def wait_out(bo, so):
        pltpu.make_async_copy(bo, out_hbm.at[pl.ds(0, _D)], so).wait()

    start_gather(0, bg0, sg0)
    start_gather(1, bg1, sg1)

    bufs = ((bg0, bo0, sg0, so0), (bg1, bo1, sg1, so1))

    def outer(i, carry):
        for b, (bg, bo, sg, so) in enumerate(bufs):
            s = i * 2 + b
            wait_gather(bg, sg)

            @pl.when(s >= 2)
            def _():
                wait_out(bo, so)

            srow = jnp.broadcast_to(s, (16,)).astype(jnp.int32)

            def d_body(d, c2):
                col = jnp.broadcast_to(d, (16,)).astype(jnp.int32)
                pe_sd = plsc.load_gather(pe_v, [srow, col])
                for j in range(_BW // 16):
                    bo[d, pl.ds(16 * j, 16)] = (
                        plsc.load_gather(bg, [biota[j], col]) + pe_sd
                    )
                return c2

            lax.fori_loop(0, _D, d_body, 0)

            @pl.when(s + 2 < _S)
            def _():
                start_gather(s + 2, bg, sg)

            start_out(s, bo, so)
        return carry

    lax.fori_loop(0, _S // 2, outer, 0)
    wait_out(bo0, so0)
    wait_out(bo1, so1)


def kernel(x, table):
    pe = _pos_encoding()
    out = _embed(x, table, pe)
    return (
        out.reshape(_S, _TD, _TB, 8, 128)
        .transpose(2, 4, 0, 1, 3)
        .reshape(_B, _S, _D)
    )


# speedup vs baseline: 1.4861x; 1.4861x over previous
import functools

import jax
import jax.numpy as jnp
from jax import lax
from jax.experimental import pallas as pl
from jax.experimental.pallas import tpu as pltpu
from jax.experimental.pallas import tpu_sc as plsc

_VOCAB = 1000000
_D = 64
_B = 4096
_S = 200

_NC, _NS = 2, 16
_NW = _NC * _NS
_BW = _B // _NW
_TB = _B // 128
_TD = _D // 8
_OUT_R = _B * _S * _D // 128


def _pos_encoding():
    pos = jnp.arange(_S, dtype=jnp.float32)[:, None]
    i = jnp.arange(_D)[None, :]
    angle_rates = 1.0 / jnp.power(10000.0, (2.0 * (i // 2)) / jnp.float32(_D))
    angle_rads = pos * angle_rates
    return jnp.where(i % 2 == 0, jnp.sin(angle_rads), jnp.cos(angle_rads)).astype(
        jnp.float32
    )


@functools.partial(
    pl.kernel,
    out_type=jax.ShapeDtypeStruct((_OUT_R, 128), jnp.float32),
    mesh=plsc.VectorSubcoreMesh(core_axis_name="c", subcore_axis_name="s"),
    scratch_types=[
        pltpu.VMEM((_BW, _S), jnp.int32),
        pltpu.VMEM((_S, _BW), jnp.int32),
        pltpu.VMEM((_S, _D), jnp.float32),
        pltpu.VMEM((_BW, _D), jnp.float32),
        pltpu.VMEM((_BW, _D), jnp.float32),
        pltpu.VMEM((_D, _BW), jnp.float32),
        pltpu.VMEM((_D, _BW), jnp.float32),
        pltpu.SemaphoreType.DMA,
        pltpu.SemaphoreType.DMA,
        pltpu.SemaphoreType.DMA,
        pltpu.SemaphoreType.DMA,
    ],
    compiler_params=pltpu.CompilerParams(
        use_tc_tiling_on_sc=False, needs_layout_passes=False
    ),
)
def _embed(
    x_hbm, table_hbm, pe_hbm, out_hbm,
    idx_v, idxt_v, pe_v, bg0, bg1, bo0, bo1, sg0, sg1, so0, so1,
):
    w = lax.axis_index("s") * _NC + lax.axis_index("c")
    pltpu.sync_copy(x_hbm.at[pl.ds(w * _BW, _BW)], idx_v)
    pltpu.sync_copy(pe_hbm, pe_v)

    lane = lax.iota(jnp.int32, 16)
    biota = [lane + 16 * j for j in range(_BW // 16)]

    @plsc.parallel_loop(0, _S, step=1)
    def tr_idx(s):
        col = jnp.broadcast_to(s, (16,)).astype(jnp.int32)
        for j in range(_BW // 16):
            idxt_v[s, pl.ds(16 * j, 16)] = plsc.load_gather(idx_v, [biota[j], col])

    def start_gather(s, bg, sg):
        pltpu.async_copy(table_hbm.at[idxt_v.at[s]], bg, sg)

    def wait_gather(bg, sg):
        pltpu.make_async_copy(table_hbm.at[pl.ds(0, _BW)], bg, sg).wait()

    def start_out(s, bo, so):
        for td in range(_TD):
            pltpu.async_copy(
                bo.at[pl.ds(td * 8, 8)],
                out_hbm.at[pl.ds(((s * _TD + td) * _TB + w) * 8, 8)],
                so,
            )

    def wait_out(bo, so):
        pltpu.make_async_copy(bo, out_hbm.at[pl.ds(0, _D)], so).wait()

    start_gather(0, bg0, sg0)
    start_gather(1, bg1, sg1)

    bufs = ((bg0, bo0, sg0, so0), (bg1, bo1, sg1, so1))

    def outer(i, carry):
        for b, (bg, bo, sg, so) in enumerate(bufs):
            s = i * 2 + b
            wait_gather(bg, sg)

            @pl.when(s >= 2)
            def _():
                wait_out(bo, so)

            srow = jnp.broadcast_to(s, (16,)).astype(jnp.int32)

            @plsc.parallel_loop(0, _D, step=1)
            def d_body(d):
                col = jnp.broadcast_to(d, (16,)).astype(jnp.int32)
                pe_sd = plsc.load_gather(pe_v, [srow, col])
                for j in range(_BW // 16):
                    bo[d, pl.ds(16 * j, 16)] = (
                        plsc.load_gather(bg, [biota[j], col]) + pe_sd
                    )

            @pl.when(s + 2 < _S)
            def _():
                start_gather(s + 2, bg, sg)

            start_out(s, bo, so)
        return carry

    lax.fori_loop(0, _S // 2, outer, 0)
    wait_out(bo0, so0)
    wait_out(bo1, so1)


def kernel(x, table):
    pe = _pos_encoding()
    out = _embed(x, table, pe)
    return (
        out.reshape(_S, _TD, _TB, 8, 128)
        .transpose(2, 4, 0, 1, 3)
        .reshape(_B, _S, _D)
    )


# speedup vs baseline: 1.5053x; 1.0130x over previous
import functools

import jax
import jax.numpy as jnp
from jax import lax
from jax.experimental import pallas as pl
from jax.experimental.pallas import tpu as pltpu
from jax.experimental.pallas import tpu_sc as plsc

_VOCAB = 1000000
_D = 64
_B = 4096
_S = 200

_NC, _NS = 2, 16
_NW = _NC * _NS
_BW = _B // _NW
_TB = _B // 128
_TD = _D // 8
_OUT_R = _B * _S * _D // 128


def _pos_encoding():
    pos = jnp.arange(_S, dtype=jnp.float32)[:, None]
    i = jnp.arange(_D)[None, :]
    angle_rates = 1.0 / jnp.power(10000.0, (2.0 * (i // 2)) / jnp.float32(_D))
    angle_rads = pos * angle_rates
    return jnp.where(i % 2 == 0, jnp.sin(angle_rads), jnp.cos(angle_rads)).astype(
        jnp.float32
    )


@functools.partial(
    pl.kernel,
    out_type=jax.ShapeDtypeStruct((_OUT_R, 128), jnp.float32),
    mesh=plsc.VectorSubcoreMesh(core_axis_name="c", subcore_axis_name="s"),
    scratch_types=[
        pltpu.VMEM((_BW, _S), jnp.int32),
        pltpu.VMEM((_S, _BW), jnp.int32),
        pltpu.VMEM((_S, _D), jnp.float32),
        pltpu.VMEM((_BW, _D), jnp.float32),
        pltpu.VMEM((_BW, _D), jnp.float32),
        pltpu.VMEM((_D, _BW), jnp.float32),
        pltpu.VMEM((_D, _BW), jnp.float32),
        pltpu.SemaphoreType.DMA,
        pltpu.SemaphoreType.DMA,
        pltpu.SemaphoreType.DMA,
        pltpu.SemaphoreType.DMA,
    ],
    compiler_params=pltpu.CompilerParams(
        use_tc_tiling_on_sc=False, needs_layout_passes=False
    ),
)
def _embed(
    x_hbm, table_hbm, pe_hbm, out_hbm,
    idx_v, idxt_v, pe_v, bg0, bg1, bo0, bo1, sg0, sg1, so0, so1,
):
    w = lax.axis_index("s") * _NC + lax.axis_index("c")
    pltpu.sync_copy(x_hbm.at[pl.ds(w * _BW, _BW)], idx_v)
    pltpu.sync_copy(pe_hbm, pe_v)

    lane = lax.iota(jnp.int32, 16)
    biota = [lane + 16 * j for j in range(_BW // 16)]

    @plsc.parallel_loop(0, _S, step=1, unroll=4)
    def tr_idx(s):
        col = jnp.broadcast_to(s, (16,)).astype(jnp.int32)
        for j in range(_BW // 16):
            idxt_v[s, pl.ds(16 * j, 16)] = plsc.load_gather(idx_v, [biota[j], col])

    def start_gather(s, bg, sg):
        pltpu.async_copy(table_hbm.at[idxt_v.at[s]], bg, sg)

    def wait_gather(bg, sg):
        pltpu.make_async_copy(table_hbm.at[pl.ds(0, _BW)], bg, sg).wait()

    def start_out(s, bo, so):
        for td in range(_TD):
            pltpu.async_copy(
                bo.at[pl.ds(td * 8, 8)],
                out_hbm.at[pl.ds(((s * _TD + td) * _TB + w) * 8, 8)],
                so,
            )

    def wait_out(bo, so):
        pltpu.make_async_copy(bo, out_hbm.at[pl.ds(0, _D)], so).wait()

    start_gather(0, bg0, sg0)
    start_gather(1, bg1, sg1)

    bufs = ((bg0, bo0, sg0, so0), (bg1, bo1, sg1, so1))

    def outer(i, carry):
        for b, (bg, bo, sg, so) in enumerate(bufs):
            s = i * 2 + b
            wait_gather(bg, sg)

            @pl.when(s >= 2)
            def _():
                wait_out(bo, so)

            srow = jnp.broadcast_to(s, (16,)).astype(jnp.int32)

            @plsc.parallel_loop(0, _D, step=1, unroll=4)
            def d_body(d):
                col = jnp.broadcast_to(d, (16,)).astype(jnp.int32)
                pe_sd = plsc.load_gather(pe_v, [srow, col])
                for j in range(_BW // 16):
                    bo[d, pl.ds(16 * j, 16)] = (
                        plsc.load_gather(bg, [biota[j], col]) + pe_sd
                    )

            @pl.when(s + 2 < _S)
            def _():
                start_gather(s + 2, bg, sg)

            start_out(s, bo, so)
        return carry

    lax.fori_loop(0, _S // 2, outer, 0)
    wait_out(bo0, so0)
    wait_out(bo1, so1)


def kernel(x, table):
    pe = _pos_encoding()
    out = _embed(x, table, pe)
    return (
        out.reshape(_S, _TD, _TB, 8, 128)
        .transpose(2, 4, 0, 1, 3)
        .reshape(_B, _S, _D)
    )


# speedup vs baseline: 2.9229x; 1.9417x over previous
import functools

import jax
import jax.numpy as jnp
from jax import lax
from jax.experimental import pallas as pl
from jax.experimental.pallas import tpu as pltpu
from jax.experimental.pallas import tpu_sc as plsc

_VOCAB = 1000000
_D = 64
_B = 4096
_S = 200

_NC, _NS = 2, 16
_NW = _NC * _NS
_BW = _B // _NW
_TB = _B // 128
_TD = _D // 8
_OUT_R = _B * _S * _D // 128


def _pos_encoding():
    pos = jnp.arange(_S, dtype=jnp.float32)[:, None]
    i = jnp.arange(_D)[None, :]
    angle_rates = 1.0 / jnp.power(10000.0, (2.0 * (i // 2)) / jnp.float32(_D))
    angle_rads = pos * angle_rates
    return jnp.where(i % 2 == 0, jnp.sin(angle_rads), jnp.cos(angle_rads)).astype(
        jnp.float32
    )


_KT = 18
_NT_W = 3906


@functools.partial(
    pl.kernel,
    out_type=jax.ShapeDtypeStruct((_VOCAB // 2, 128), jnp.float32),
    mesh=plsc.VectorSubcoreMesh(core_axis_name="c", subcore_axis_name="s"),
    scratch_types=[
        pltpu.VMEM((_KT, 8, _D), jnp.float32),
        pltpu.VMEM((_KT, 8, _D), jnp.float32),
        pltpu.VMEM((_KT * 4, 128), jnp.float32),
        pltpu.VMEM((_KT * 4, 128), jnp.float32),
        pltpu.SemaphoreType.DMA,
        pltpu.SemaphoreType.DMA,
        pltpu.SemaphoreType.DMA,
        pltpu.SemaphoreType.DMA,
    ],
    compiler_params=pltpu.CompilerParams(
        use_tc_tiling_on_sc=True, needs_layout_passes=False
    ),
)
def _repack(t4_hbm, out_hbm, bi0, bi1, bp0, bp1, si0, si1, so0, so1):
    w = lax.axis_index("s") * _NC + lax.axis_index("c")
    base = w * _NT_W

    def start_in(g, bi, si):
        pltpu.async_copy(t4_hbm.at[pl.ds(base + g * _KT, _KT)], bi, si)

    def wait_in(bi, si):
        pltpu.make_async_copy(t4_hbm.at[pl.ds(0, _KT)], bi, si).wait()

    def start_o(g, bp, so):
        pltpu.async_copy(bp, out_hbm.at[pl.ds((base + g * _KT) * 4, _KT * 4)], so)

    def wait_o(bp, so):
        pltpu.make_async_copy(bp, out_hbm.at[pl.ds(0, _KT * 4)], so).wait()

    start_in(0, bi0, si0)
    start_in(1, bi1, si1)
    bufs = ((bi0, bp0, si0, so0), (bi1, bp1, si1, so1))

    def outer(i, carry):
        for b, (bi, bp, si, so) in enumerate(bufs):
            g = i * 2 + b
            wait_in(bi, si)

            @pl.when(g >= 2)
            def _():
                wait_o(bp, so)

            @plsc.parallel_loop(0, _KT * 8, step=1, unroll=4)
            def row_body(r):
                t = r // 8
                rr = r - t * 8
                half = (r & 1) * _D
                for c in range(_D // 16):
                    bp[r // 2, pl.ds(half + 16 * c, 16)] = bi[t, rr, pl.ds(16 * c, 16)]

            @pl.when(g + 2 < _NT_W // _KT)
            def _():
                start_in(g + 2, bi, si)

            start_o(g, bp, so)
        return carry

    lax.fori_loop(0, _NT_W // _KT // 2, outer, 0)

    last = _NT_W // _KT - 1
    wait_in(bi0, si0)
    wait_o(bp0, so0)

    @plsc.parallel_loop(0, _KT * 8, step=1, unroll=4)
    def last_body(r):
        t = r // 8
        rr = r - t * 8
        half = (r & 1) * _D
        for c in range(_D // 16):
            bp0[r // 2, pl.ds(half + 16 * c, 16)] = bi0[t, rr, pl.ds(16 * c, 16)]

    start_o(last, bp0, so0)
    wait_o(bp0, so0)
    wait_o(bp1, so1)

    @pl.when(w == 0)
    def _():
        pltpu.async_copy(t4_hbm.at[pl.ds(32 * _NT_W, 8)], bi0.at[pl.ds(0, 8)], si0)
        pltpu.make_async_copy(
            t4_hbm.at[pl.ds(0, 8)], bi0.at[pl.ds(0, 8)], si0
        ).wait()

        @plsc.parallel_loop(0, 64, step=1, unroll=4)
        def tail_body(r):
            t = r // 8
            rr = r - t * 8
            half = (r & 1) * _D
            for c in range(_D // 16):
                bp0[r // 2, pl.ds(half + 16 * c, 16)] = bi0[t, rr, pl.ds(16 * c, 16)]

        pltpu.async_copy(
            bp0.at[pl.ds(0, 32)], out_hbm.at[pl.ds(32 * _NT_W * 4, 32)], so0
        )
        pltpu.make_async_copy(
            bp0.at[pl.ds(0, 32)], out_hbm.at[pl.ds(0, 32)], so0
        ).wait()


@functools.partial(
    pl.kernel,
    out_type=jax.ShapeDtypeStruct((_OUT_R, 128), jnp.float32),
    mesh=plsc.VectorSubcoreMesh(core_axis_name="c", subcore_axis_name="s"),
    scratch_types=[
        pltpu.VMEM((_BW, _S), jnp.int32),
        pltpu.VMEM((_S, _BW), jnp.int32),
        pltpu.VMEM((_S, _D), jnp.float32),
        pltpu.VMEM((_BW, _D), jnp.float32),
        pltpu.VMEM((_BW, _D), jnp.float32),
        pltpu.VMEM((_D, 132), jnp.float32),
        pltpu.VMEM((_D, 132), jnp.float32),
        pltpu.SemaphoreType.DMA,
        pltpu.SemaphoreType.DMA,
        pltpu.SemaphoreType.DMA,
        pltpu.SemaphoreType.DMA,
    ],
    compiler_params=pltpu.CompilerParams(
        use_tc_tiling_on_sc=False, needs_layout_passes=False
    ),
)
def _embed(
    x_hbm, table_hbm, pe_hbm, out_hbm,
    idx_v, idxt_v, pe_v, bg0, bg1, bo0, bo1, sg0, sg1, so0, so1,
):
    w = lax.axis_index("s") * _NC + lax.axis_index("c")
    pltpu.sync_copy(x_hbm.at[pl.ds(w * _BW, _BW)], idx_v)
    pltpu.sync_copy(pe_hbm, pe_v)

    lane = lax.iota(jnp.int32, 16)
    biota = [lane + 16 * j for j in range(_BW // 16)]
    diota = [lane + 16 * c for c in range(_D // 16)]

    @plsc.parallel_loop(0, _S, step=1, unroll=4)
    def tr_idx(s):
        col = jnp.broadcast_to(s, (16,)).astype(jnp.int32)
        for j in range(_BW // 16):
            idxt_v[s, pl.ds(16 * j, 16)] = plsc.load_gather(idx_v, [biota[j], col])

    def start_gather(s, bg, sg):
        pltpu.async_copy(table_hbm.at[idxt_v.at[s]], bg, sg)

    def wait_gather(bg, sg):
        pltpu.make_async_copy(table_hbm.at[pl.ds(0, _BW)], bg, sg).wait()

    def start_out(s, bo, so):
        for td in range(_TD):
            pltpu.async_copy(
                bo.at[pl.ds(td * 8, 8), pl.ds(0, 128)],
                out_hbm.at[pl.ds(((s * _TD + td) * _TB + w) * 8, 8)],
                so,
            )

    def wait_out(bo, so):
        pltpu.make_async_copy(
            bo.at[pl.ds(0, _D), pl.ds(0, 128)], out_hbm.at[pl.ds(0, _D)], so
        ).wait()

    start_gather(0, bg0, sg0)
    start_gather(1, bg1, sg1)

    bufs = ((bg0, bo0, sg0, so0), (bg1, bo1, sg1, so1))

    def outer(i, carry):
        for b, (bg, bo, sg, so) in enumerate(bufs):
            s = i * 2 + b
            wait_gather(bg, sg)

            @pl.when(s >= 2)
            def _():
                wait_out(bo, so)

            pe_s = [pe_v[s, pl.ds(16 * c, 16)] for c in range(_D // 16)]

            @plsc.parallel_loop(0, _BW, step=1, unroll=4)
            def b_body(b):
                colb = jnp.broadcast_to(b, (16,)).astype(jnp.int32)
                for c in range(_D // 16):
                    v = bg[b, pl.ds(16 * c, 16)] + pe_s[c]
                    plsc.store_scatter(bo, [diota[c], colb], v)

            @pl.when(s + 2 < _S)
            def _():
                start_gather(s + 2, bg, sg)

            start_out(s, bo, so)
        return carry

    lax.fori_loop(0, _S // 2, outer, 0)
    wait_out(bo0, so0)
    wait_out(bo1, so1)


def kernel(x, table):
    pe = _pos_encoding()
    t64 = _repack(table.reshape(_VOCAB // 8, 8, _D)).reshape(_VOCAB, _D)
    out = _embed(x, t64, pe)
    return (
        out.reshape(_S, _TD, _TB, 8, 128)
        .transpose(2, 4, 0, 1, 3)
        .reshape(_B, _S, _D)
    )
